# baseline (speedup 1.0000x reference)
import jax
import jax.numpy as jnp
from jax.experimental import pallas as pl
from jax.experimental.pallas import tpu as pltpu


def _gcn_body(x_ref, a_ref, w1_ref, b1_ref, w2_ref, b2_ref, h2_ref,
              s1_ref, s2_ref):
    p = pl.program_id(0)
    i = pl.program_id(1)
    bm = x_ref.shape[0]

    @pl.when(p == 0)
    def _():
        s1_ref[pl.ds(i * bm, bm), :] = jnp.dot(
            x_ref[...], w1_ref[...], preferred_element_type=jnp.float32)

    @pl.when(p == 1)
    def _():
        h = jnp.dot(a_ref[...], s1_ref[...],
                    preferred_element_type=jnp.float32) + b1_ref[...]
        h = jnp.maximum(h, 0.0)
        s2_ref[pl.ds(i * bm, bm), :] = jnp.dot(
            h, w2_ref[...], preferred_element_type=jnp.float32)

    @pl.when(p == 2)
    def _():
        h2_ref[...] = jnp.dot(a_ref[...], s2_ref[...],
                              preferred_element_type=jnp.float32) + b2_ref[...]


def _logsoftmax_body(h_ref, o_ref):
    h = h_ref[...]
    m = jnp.max(h, axis=0, keepdims=True)
    lse = jnp.log(jnp.sum(jnp.exp(h - m), axis=0, keepdims=True)) + m
    o_ref[...] = h - lse


def kernel(features, adj_matrix, W1, b1, W2, b2):
    n, nin = features.shape
    nhid = W1.shape[1]
    nout = W2.shape[1]
    bm = 200
    nb = n // bm
    b1r = b1.reshape(1, nhid)
    b2r = b2.reshape(1, nout)

    h2 = pl.pallas_call(
        _gcn_body,
        grid=(3, nb),
        in_specs=[
            pl.BlockSpec((bm, nin), lambda p, i: (jnp.where(p == 0, i, 0), 0)),
            pl.BlockSpec((bm, n), lambda p, i: (jnp.where(p == 0, 0, i), 0)),
            pl.BlockSpec((nin, nhid), lambda p, i: (0, 0)),
            pl.BlockSpec((1, nhid), lambda p, i: (0, 0)),
            pl.BlockSpec((nhid, nout), lambda p, i: (0, 0)),
            pl.BlockSpec((1, nout), lambda p, i: (0, 0)),
        ],
        out_specs=pl.BlockSpec((bm, nout),
                               lambda p, i: (jnp.where(p == 2, i, 0), 0)),
        out_shape=jax.ShapeDtypeStruct((n, nout), jnp.float32),
        scratch_shapes=[
            pltpu.VMEM((n, nhid), jnp.float32),
            pltpu.VMEM((n, nout), jnp.float32),
        ],
    )(features, adj_matrix, W1, b1r, W2, b2r)

    out = pl.pallas_call(
        _logsoftmax_body,
        out_shape=jax.ShapeDtypeStruct((n, nout), jnp.float32),
    )(h2)
    return out


# speedup vs baseline: 1.0403x; 1.0403x over previous
import jax
import jax.numpy as jnp
from jax.experimental import pallas as pl
from jax.experimental.pallas import tpu as pltpu


def _gcn_body(x_ref, a_ref, w1_ref, b1_ref, w2_ref, b2_ref, h2_ref,
              s1_ref, s2_ref):
    p = pl.program_id(0)
    i = pl.program_id(1)
    bm = x_ref.shape[0]

    @pl.when(p == 0)
    def _():
        s1_ref[pl.ds(i * bm, bm), :] = jnp.dot(
            x_ref[...], w1_ref[...], preferred_element_type=jnp.float32)

    @pl.when(p == 1)
    def _():
        h = jnp.dot(a_ref[...], s1_ref[...],
                    preferred_element_type=jnp.float32) + b1_ref[...]
        h = jnp.maximum(h, 0.0)
        s2_ref[pl.ds(i * bm, bm), :] = jnp.dot(
            h, w2_ref[...], preferred_element_type=jnp.float32)

    @pl.when(p == 2)
    def _():
        h2_ref[...] = jnp.dot(a_ref[...], s2_ref[...],
                              preferred_element_type=jnp.float32) + b2_ref[...]


def _logsoftmax_body(h_ref, o_ref):
    h = h_ref[...]
    m = jnp.max(h, axis=0, keepdims=True)
    lse = jnp.log(jnp.sum(jnp.exp(h - m), axis=0, keepdims=True)) + m
    o_ref[...] = h - lse


def kernel(features, adj_matrix, W1, b1, W2, b2):
    n, nin = features.shape
    nhid = W1.shape[1]
    nout = W2.shape[1]
    bm = 400
    nb = n // bm
    b1r = b1.reshape(1, nhid)
    b2r = b2.reshape(1, nout)

    h2 = pl.pallas_call(
        _gcn_body,
        grid=(3, nb),
        in_specs=[
            pl.BlockSpec((bm, nin), lambda p, i: (jnp.where(p == 0, i, 0), 0)),
            pl.BlockSpec((bm, n), lambda p, i: (jnp.where(p == 0, 0, i), 0)),
            pl.BlockSpec((nin, nhid), lambda p, i: (0, 0)),
            pl.BlockSpec((1, nhid), lambda p, i: (0, 0)),
            pl.BlockSpec((nhid, nout), lambda p, i: (0, 0)),
            pl.BlockSpec((1, nout), lambda p, i: (0, 0)),
        ],
        out_specs=pl.BlockSpec((bm, nout),
                               lambda p, i: (jnp.where(p == 2, i, 0), 0)),
        out_shape=jax.ShapeDtypeStruct((n, nout), jnp.float32),
        scratch_shapes=[
            pltpu.VMEM((n, nhid), jnp.float32),
            pltpu.VMEM((n, nout), jnp.float32),
        ],
    )(features, adj_matrix, W1, b1r, W2, b2r)

    out = pl.pallas_call(
        _logsoftmax_body,
        out_shape=jax.ShapeDtypeStruct((n, nout), jnp.float32),
    )(h2)
    return out


# speedup vs baseline: 1.0977x; 1.0552x over previous
import jax
import jax.numpy as jnp
from jax.experimental import pallas as pl
from jax.experimental.pallas import tpu as pltpu


def _gcn_body(x_ref, a_ref, w1_ref, b1_ref, w2_ref, b2_ref, out_ref,
              s1_ref, s2_ref):
    p = pl.program_id(0)
    i = pl.program_id(1)
    nb = pl.num_programs(1)
    bm = a_ref.shape[0]

    @pl.when(jnp.logical_and(p == 0, i == 0))
    def _():
        s1_ref[...] = jnp.dot(x_ref[...], w1_ref[...],
                              preferred_element_type=jnp.float32)

    @pl.when(p == 0)
    def _():
        h = jnp.dot(a_ref[...], s1_ref[...],
                    preferred_element_type=jnp.float32) + b1_ref[...]
        h = jnp.maximum(h, 0.0)
        s2_ref[pl.ds(i * bm, bm), :] = jnp.dot(
            h, w2_ref[...], preferred_element_type=jnp.float32)

    @pl.when(p == 1)
    def _():
        out_ref[pl.ds(i * bm, bm), :] = jnp.dot(
            a_ref[...], s2_ref[...],
            preferred_element_type=jnp.float32) + b2_ref[...]

    @pl.when(jnp.logical_and(p == 1, i == nb - 1))
    def _():
        h2 = out_ref[...]
        m = jnp.max(h2, axis=0, keepdims=True)
        lse = jnp.log(jnp.sum(jnp.exp(h2 - m), axis=0, keepdims=True)) + m
        out_ref[...] = h2 - lse


def kernel(features, adj_matrix, W1, b1, W2, b2):
    n, nin = features.shape
    nhid = W1.shape[1]
    nout = W2.shape[1]
    bm = 400
    nb = n // bm
    b1r = b1.reshape(1, nhid)
    b2r = b2.reshape(1, nout)

    out = pl.pallas_call(
        _gcn_body,
        grid=(2, nb),
        in_specs=[
            pl.BlockSpec((n, nin), lambda p, i: (0, 0)),
            pl.BlockSpec((bm, n), lambda p, i: (i, 0)),
            pl.BlockSpec((nin, nhid), lambda p, i: (0, 0)),
            pl.BlockSpec((1, nhid), lambda p, i: (0, 0)),
            pl.BlockSpec((nhid, nout), lambda p, i: (0, 0)),
            pl.BlockSpec((1, nout), lambda p, i: (0, 0)),
        ],
        out_specs=pl.BlockSpec((n, nout), lambda p, i: (0, 0)),
        out_shape=jax.ShapeDtypeStruct((n, nout), jnp.float32),
        scratch_shapes=[
            pltpu.VMEM((n, nhid), jnp.float32),
            pltpu.VMEM((n, nout), jnp.float32),
        ],
    )(features, adj_matrix, W1, b1r, W2, b2r)
    return out
